# baseline (speedup 1.0000x reference)
import functools

import jax
import jax.numpy as jnp
from jax import lax
from jax.experimental import pallas as pl
from jax.experimental.pallas import tpu as pltpu
from jax.experimental.pallas import tpu_sc as plsc

N = 10000
E = 320000
D = 128
DH = D // 2

NC = 2
NS = 16
CHUNK = 128
NCHUNK = E // CHUNK
TMAX = -(-NCHUNK // NS)
NRING = 3
NSETS = 2
NB = NRING * NSETS
QMAX = -(-TMAX // NRING)
Q2MAX = (QMAX + 2 + 1) // 2

NBLK = N // CHUNK
NREM = N - NBLK * CHUNK
ZITER = -(-NBLK // NS)


def _sc_agg_body(x_hbm, ei_hbm, agg_out, deg_out, *refs):
    idx_b = refs[0:NB]
    rows_b = refs[NB:2 * NB]
    ones_v = refs[2 * NB]
    zdeg_v = refs[2 * NB + 1]
    acc = refs[2 * NB + 2]
    dacc = refs[2 * NB + 3]
    base = 2 * NB + 4
    i_sem = refs[base: base + NB]
    g_sem = refs[base + NB: base + 2 * NB]
    s_sem = refs[base + 2 * NB: base + 3 * NB]
    d_sem = refs[base + 3 * NB: base + 4 * NB]

    c = lax.axis_index("c")
    s = lax.axis_index("s")
    _VEC0 = jnp.zeros((16,), jnp.float32)
    _VEC1 = jnp.ones((16,), jnp.float32)

    def _fill(i, _):
        for j in range(DH // 16):
            rows_b[0][i, pl.ds(j * 16, 16)] = _VEC0
        return 0

    def _fill_small(i, _):
        ones_v[pl.ds(i * 16, 16)] = _VEC1
        zdeg_v[pl.ds(i * 16, 16)] = _VEC0
        return 0

    lax.fori_loop(0, CHUNK, _fill, 0)
    lax.fori_loop(0, CHUNK // 16, _fill_small, 0)

    def _zero_blk(t, _):
        b = s + NS * t

        @pl.when(b < NBLK)
        def _():
            r0 = b * CHUNK
            pltpu.sync_copy(rows_b[0], acc.at[pl.ds(r0, CHUNK), :])
            pltpu.sync_copy(zdeg_v, dacc.at[pl.ds(r0, CHUNK)])

        return 0

    lax.fori_loop(0, ZITER, _zero_blk, 0)

    @pl.when(s == NBLK % NS)
    def _():
        pltpu.sync_copy(rows_b[0].at[pl.ds(0, NREM), :], acc.at[pl.ds(NBLK * CHUNK, NREM), :])
        pltpu.sync_copy(zdeg_v.at[pl.ds(0, NREM)], dacc.at[pl.ds(NBLK * CHUNK, NREM)])

    plsc.subcore_barrier()

    def _valid(u):
        return s + NS * u < NCHUNK

    def _e0(u):
        return (s + NS * u) * CHUNK

    src_off = c * N

    def _ring(q2, _):
        for dq in range(NSETS):
            q = NSETS * q2 + dq
            boff = dq * NRING

            for k in range(NRING):
                kb = boff + k

                @pl.when((q >= 2) & _valid(NRING * (q - 2) + k))
                def _(kb=kb):
                    pltpu.make_async_copy(rows_b[kb], acc.at[idx_b[kb].at[1]], s_sem[kb]).wait()

                    @pl.when(c == 0)
                    def _():
                        pltpu.make_async_copy(ones_v, dacc.at[idx_b[kb].at[1]], d_sem[kb]).wait()

            for k in range(NRING):
                kb = boff + k
                u = NRING * q + k

                @pl.when((q < QMAX) & _valid(u))
                def _(kb=kb, u=u):
                    pltpu.async_copy(ei_hbm.at[:, pl.ds(_e0(u), CHUNK)], idx_b[kb], i_sem[kb])

            for k in range(NRING):
                kb = boff + k
                u = NRING * q + k

                @pl.when((q < QMAX) & _valid(u))
                def _(kb=kb):
                    pltpu.make_async_copy(ei_hbm.at[:, pl.ds(0, CHUNK)], idx_b[kb], i_sem[kb]).wait()
                    for j in range(CHUNK // 16):
                        idx_b[kb][0, pl.ds(j * 16, 16)] = (
                            idx_b[kb][0, pl.ds(j * 16, 16)] + src_off
                        )
                    pltpu.async_copy(x_hbm.at[idx_b[kb].at[0]], rows_b[kb], g_sem[kb])

            for k in range(NRING):
                kb = boff + k
                u = NRING * q + k

                @pl.when((q < QMAX) & _valid(u))
                def _(kb=kb):
                    pltpu.make_async_copy(x_hbm.at[idx_b[kb].at[0]], rows_b[kb], g_sem[kb]).wait()
                    pltpu.async_copy(rows_b[kb], acc.at[idx_b[kb].at[1]], s_sem[kb], add=True)

                    @pl.when(c == 0)
                    def _():
                        pltpu.async_copy(ones_v, dacc.at[idx_b[kb].at[1]], d_sem[kb], add=True)

        return 0

    lax.fori_loop(0, Q2MAX, _ring, 0)
    plsc.subcore_barrier()

    def _out_blk(t, _):
        b = s + NS * t

        @pl.when(b < NBLK)
        def _():
            r0 = b * CHUNK
            pltpu.sync_copy(acc.at[pl.ds(r0, CHUNK), :], rows_b[0])
            pltpu.sync_copy(rows_b[0], agg_out.at[c, pl.ds(r0, CHUNK), :])

            @pl.when(c == 0)
            def _():
                pltpu.sync_copy(dacc.at[pl.ds(r0, CHUNK)], zdeg_v)
                pltpu.sync_copy(zdeg_v, deg_out.at[pl.ds(r0, CHUNK)])

        return 0

    lax.fori_loop(0, ZITER, _out_blk, 0)

    @pl.when(s == NBLK % NS)
    def _():
        r0 = NBLK * CHUNK
        pltpu.sync_copy(acc.at[pl.ds(r0, NREM), :], rows_b[0].at[pl.ds(0, NREM), :])
        pltpu.sync_copy(rows_b[0].at[pl.ds(0, NREM), :], agg_out.at[c, pl.ds(r0, NREM), :])

        @pl.when(c == 0)
        def _():
            pltpu.sync_copy(dacc.at[pl.ds(r0, NREM)], zdeg_v.at[pl.ds(0, NREM)])
            pltpu.sync_copy(zdeg_v.at[pl.ds(0, NREM)], deg_out.at[pl.ds(r0, NREM)])


_sc_agg = functools.partial(
    pl.kernel,
    out_type=[
        jax.ShapeDtypeStruct((NC, N, DH), jnp.float32),
        jax.ShapeDtypeStruct((N,), jnp.float32),
    ],
    mesh=plsc.VectorSubcoreMesh(core_axis_name="c", subcore_axis_name="s"),
    compiler_params=pltpu.CompilerParams(use_tc_tiling_on_sc=False),
    scratch_types=(
        [pltpu.VMEM((2, CHUNK), jnp.int32) for _ in range(NB)]
        + [pltpu.VMEM((CHUNK, DH), jnp.float32) for _ in range(NB)]
        + [
            pltpu.VMEM((CHUNK,), jnp.float32),
            pltpu.VMEM((CHUNK,), jnp.float32),
            pltpu.VMEM_SHARED((N, DH), jnp.float32),
            pltpu.VMEM_SHARED((N,), jnp.float32),
        ]
        + [pltpu.SemaphoreType.DMA for _ in range(4 * NB)]
    ),
)(_sc_agg_body)


RB = 2000


def _combine_body(x_ref, w_ref, a_ref, d_ref, o_ref):
    x = x_ref[...]
    w = w_ref[...]
    a = jnp.concatenate([a_ref[0], a_ref[1]], axis=1)
    d = d_ref[...]
    neigh = a / jnp.maximum(d, 1.0)
    out = jnp.dot(x, w, preferred_element_type=jnp.float32) + neigh
    nrm = jnp.sqrt(jnp.sum(out * out, axis=1, keepdims=True))
    o_ref[...] = out / jnp.maximum(nrm, 1e-12)


def _combine(x, w, agg, deg):
    return pl.pallas_call(
        _combine_body,
        grid=(N // RB,),
        in_specs=[
            pl.BlockSpec((RB, D), lambda i: (i, 0)),
            pl.BlockSpec((D, D), lambda i: (0, 0)),
            pl.BlockSpec((NC, RB, DH), lambda i: (0, i, 0)),
            pl.BlockSpec((RB, 1), lambda i: (i, 0)),
        ],
        out_specs=pl.BlockSpec((RB, D), lambda i: (i, 0)),
        out_shape=jax.ShapeDtypeStruct((N, D), jnp.float32),
    )(x, w, agg, deg)


@jax.jit
def kernel(x, edge_index, w):
    x_split = jnp.concatenate([x[:, :DH], x[:, DH:]], axis=0)
    agg, deg = _sc_agg(x_split, edge_index)
    return _combine(x, w, agg, deg.reshape(N, 1))


# speedup vs baseline: 1.1069x; 1.1069x over previous
import functools

import jax
import jax.numpy as jnp
from jax import lax
from jax.experimental import pallas as pl
from jax.experimental.pallas import tpu as pltpu
from jax.experimental.pallas import tpu_sc as plsc

N = 10000
E = 320000
D = 128

NC = 2
NS = 16
CHUNK = 64
NCHUNK = E // CHUNK
CH_PER_CORE = NCHUNK // NC
TMAX = -(-CH_PER_CORE // NS)
NRING = 3
NSETS = 2
NB = NRING * NSETS
QMAX = -(-TMAX // NRING)
Q2MAX = (QMAX + 2 + 1) // 2

BBLK = 64
NBLK = N // BBLK
NREM = N - NBLK * BBLK
ZITER = -(-NBLK // NS)


def _sc_agg_body(x_hbm, ei_hbm, agg_out, deg_out, *refs):
    idx_b = refs[0:NB]
    rows_b = refs[NB:2 * NB]
    ones_v = refs[2 * NB]
    zdeg_v = refs[2 * NB + 1]
    acc = refs[2 * NB + 2]
    dacc = refs[2 * NB + 3]
    base = 2 * NB + 4
    i_sem = refs[base: base + NB]
    g_sem = refs[base + NB: base + 2 * NB]
    s_sem = refs[base + 2 * NB: base + 3 * NB]
    d_sem = refs[base + 3 * NB: base + 4 * NB]

    c = lax.axis_index("c")
    s = lax.axis_index("s")
    _VEC0 = jnp.zeros((16,), jnp.float32)
    _VEC1 = jnp.ones((16,), jnp.float32)

    def _fill(i, _):
        for j in range(D // 16):
            rows_b[0][i, pl.ds(j * 16, 16)] = _VEC0
        return 0

    def _fill_small(i, _):
        ones_v[pl.ds(i * 16, 16)] = _VEC1
        zdeg_v[pl.ds(i * 16, 16)] = _VEC0
        return 0

    lax.fori_loop(0, CHUNK, _fill, 0)
    lax.fori_loop(0, CHUNK // 16, _fill_small, 0)

    def _zero_blk(t, _):
        b = s + NS * t

        @pl.when(b < NBLK)
        def _():
            r0 = b * BBLK
            pltpu.sync_copy(rows_b[0], acc.at[pl.ds(r0, BBLK), :])
            pltpu.sync_copy(zdeg_v, dacc.at[pl.ds(r0, BBLK)])

        return 0

    lax.fori_loop(0, ZITER, _zero_blk, 0)

    @pl.when(s == NBLK % NS)
    def _():
        pltpu.sync_copy(rows_b[0].at[pl.ds(0, NREM), :], acc.at[pl.ds(NBLK * BBLK, NREM), :])
        pltpu.sync_copy(zdeg_v.at[pl.ds(0, NREM)], dacc.at[pl.ds(NBLK * BBLK, NREM)])

    plsc.subcore_barrier()

    def _valid(u):
        return s + NS * u < CH_PER_CORE

    def _e0(u):
        return (c * CH_PER_CORE + s + NS * u) * CHUNK

    def _ring(q2, _):
        for dq in range(NSETS):
            q = NSETS * q2 + dq
            boff = dq * NRING

            for k in range(NRING):
                kb = boff + k

                @pl.when((q >= 2) & _valid(NRING * (q - 2) + k))
                def _(kb=kb):
                    pltpu.make_async_copy(rows_b[kb], acc.at[idx_b[kb].at[1]], s_sem[kb]).wait()
                    pltpu.make_async_copy(ones_v, dacc.at[idx_b[kb].at[1]], d_sem[kb]).wait()

            for k in range(NRING):
                kb = boff + k
                u = NRING * q + k

                @pl.when((q < QMAX) & _valid(u))
                def _(kb=kb, u=u):
                    pltpu.async_copy(ei_hbm.at[:, pl.ds(_e0(u), CHUNK)], idx_b[kb], i_sem[kb])

            for k in range(NRING):
                kb = boff + k
                u = NRING * q + k

                @pl.when((q < QMAX) & _valid(u))
                def _(kb=kb):
                    pltpu.make_async_copy(ei_hbm.at[:, pl.ds(0, CHUNK)], idx_b[kb], i_sem[kb]).wait()
                    pltpu.async_copy(x_hbm.at[idx_b[kb].at[0]], rows_b[kb], g_sem[kb])

            for k in range(NRING):
                kb = boff + k
                u = NRING * q + k

                @pl.when((q < QMAX) & _valid(u))
                def _(kb=kb):
                    pltpu.make_async_copy(x_hbm.at[idx_b[kb].at[0]], rows_b[kb], g_sem[kb]).wait()
                    pltpu.async_copy(rows_b[kb], acc.at[idx_b[kb].at[1]], s_sem[kb], add=True)
                    pltpu.async_copy(ones_v, dacc.at[idx_b[kb].at[1]], d_sem[kb], add=True)

        return 0

    lax.fori_loop(0, Q2MAX, _ring, 0)
    plsc.subcore_barrier()

    def _out_blk(t, _):
        b = s + NS * t

        @pl.when(b < NBLK)
        def _():
            r0 = b * BBLK
            pltpu.sync_copy(acc.at[pl.ds(r0, BBLK), :], rows_b[0])
            pltpu.sync_copy(rows_b[0], agg_out.at[c, pl.ds(r0, BBLK), :])
            pltpu.sync_copy(dacc.at[pl.ds(r0, BBLK)], zdeg_v)
            pltpu.sync_copy(zdeg_v, deg_out.at[pl.ds(c * N + r0, BBLK)])

        return 0

    lax.fori_loop(0, ZITER, _out_blk, 0)

    @pl.when(s == NBLK % NS)
    def _():
        r0 = NBLK * BBLK
        pltpu.sync_copy(acc.at[pl.ds(r0, NREM), :], rows_b[0].at[pl.ds(0, NREM), :])
        pltpu.sync_copy(rows_b[0].at[pl.ds(0, NREM), :], agg_out.at[c, pl.ds(r0, NREM), :])
        pltpu.sync_copy(dacc.at[pl.ds(r0, NREM)], zdeg_v.at[pl.ds(0, NREM)])
        pltpu.sync_copy(zdeg_v.at[pl.ds(0, NREM)], deg_out.at[pl.ds(c * N + r0, NREM)])


_sc_agg = functools.partial(
    pl.kernel,
    out_type=[
        jax.ShapeDtypeStruct((NC, N, D), jnp.float32),
        jax.ShapeDtypeStruct((NC * N,), jnp.float32),
    ],
    mesh=plsc.VectorSubcoreMesh(core_axis_name="c", subcore_axis_name="s"),
    compiler_params=pltpu.CompilerParams(use_tc_tiling_on_sc=False),
    scratch_types=(
        [pltpu.VMEM((2, CHUNK), jnp.int32) for _ in range(NB)]
        + [pltpu.VMEM((CHUNK, D), jnp.float32) for _ in range(NB)]
        + [
            pltpu.VMEM((CHUNK,), jnp.float32),
            pltpu.VMEM((CHUNK,), jnp.float32),
            pltpu.VMEM_SHARED((N, D), jnp.float32),
            pltpu.VMEM_SHARED((N,), jnp.float32),
        ]
        + [pltpu.SemaphoreType.DMA for _ in range(4 * NB)]
    ),
)(_sc_agg_body)


RB = 2000


def _combine_body(x_ref, w_ref, a_ref, d_ref, o_ref):
    x = x_ref[...]
    w = w_ref[...]
    a = a_ref[0] + a_ref[1]
    d = d_ref[0] + d_ref[1]
    neigh = a / jnp.maximum(d, 1.0)
    out = jnp.dot(x, w, preferred_element_type=jnp.float32) + neigh
    nrm = jnp.sqrt(jnp.sum(out * out, axis=1, keepdims=True))
    o_ref[...] = out / jnp.maximum(nrm, 1e-12)


def _combine(x, w, agg, deg):
    return pl.pallas_call(
        _combine_body,
        grid=(N // RB,),
        in_specs=[
            pl.BlockSpec((RB, D), lambda i: (i, 0)),
            pl.BlockSpec((D, D), lambda i: (0, 0)),
            pl.BlockSpec((NC, RB, D), lambda i: (0, i, 0)),
            pl.BlockSpec((NC, RB, 1), lambda i: (0, i, 0)),
        ],
        out_specs=pl.BlockSpec((RB, D), lambda i: (i, 0)),
        out_shape=jax.ShapeDtypeStruct((N, D), jnp.float32),
    )(x, w, agg, deg)


@jax.jit
def kernel(x, edge_index, w):
    agg, deg = _sc_agg(x, edge_index)
    return _combine(x, w, agg, deg.reshape(NC, N, 1))
